# baseline (speedup 1.0000x reference)
import jax
import jax.numpy as jnp
from jax import lax
from jax.experimental import pallas as pl
from jax.experimental.pallas import tpu as pltpu
from jax.experimental.pallas import tpu_sc as plsc

_D = 32
_NC, _NS = 2, 16
_NW = _NC * _NS
_H = 8
_LANES = 128
_SPW = 4


def _gather_body(ids_hbm, offs_hbm, tab_hbm, out_hbm,
                 idx_v, offs_v, gbuf, tbuf,
                 gs0, gs1, gs2, gs3, gs4, gs5, gs6, gs7, wsem):
    wid = lax.axis_index("s") * _NC + lax.axis_index("c")
    gsems = (gs0, gs1, gs2, gs3, gs4, gs5, gs6, gs7)
    n_tg = ids_hbm.shape[0] // 4

    pltpu.sync_copy(offs_hbm, offs_v)

    def _super(s, carry):
        S = wid * _SPW + s
        b = S // n_tg
        tg = S % n_tg
        pltpu.sync_copy(ids_hbm.at[S], idx_v)
        for h in range(_H):
            off = offs_v[h, :]
            for k in range(_LANES // 16):
                sl = (h, pl.ds(k * 16, 16))
                idx_v[sl] = idx_v[sl] + off
        for h in range(_H):
            pltpu.async_copy(tab_hbm.at[idx_v.at[h]], gbuf.at[h], gsems[h])
        iota = lax.iota(jnp.int32, 16)
        for h in range(_H):
            pltpu.make_async_copy(tab_hbm.at[idx_v.at[0]], gbuf.at[h],
                                  gsems[h]).wait()

            def _transp(d, c):
                hv = jnp.full((16,), h, jnp.int32)
                dv = jnp.full((16,), d, jnp.int32)
                for g in range(_LANES // 16):
                    rows = iota + (g * 16)
                    val = plsc.load_gather(gbuf, [hv, rows, dv])
                    tbuf[h, d, pl.ds(g * 16, 16)] = val
                return c
            lax.fori_loop(0, _D, _transp, 0)
            for dg in range(4):
                pltpu.async_copy(tbuf.at[h, pl.ds(dg * 8, 8)],
                                 out_hbm.at[b, h, dg, tg], wsem)
        for _ in range(_H * 4):
            pltpu.make_async_copy(tbuf.at[0, pl.ds(0, 8)],
                                  out_hbm.at[0, 0, 0, 0], wsem).wait()
        return carry

    lax.fori_loop(0, _SPW, _super, 0)


def kernel(input_ids, embedding, offsets):
    B, T, H = input_ids.shape
    n_tg = T // _LANES
    n_super = B * n_tg

    ids3 = input_ids.reshape(B, n_tg, _LANES, H).transpose(0, 1, 3, 2)
    ids3 = ids3.reshape(n_super, H, _LANES).astype(jnp.int32)
    offs16 = jnp.tile(offsets.astype(jnp.int32).reshape(H, 1), (1, 16))

    mesh = plsc.VectorSubcoreMesh(core_axis_name="c", subcore_axis_name="s",
                                  num_cores=_NC, num_subcores=_NS)
    run = pl.kernel(
        _gather_body,
        out_type=jax.ShapeDtypeStruct((B, H, _D // 8, n_tg, 8, _LANES),
                                      jnp.float32),
        mesh=mesh,
        scratch_types=[
            pltpu.VMEM((_H, _LANES), jnp.int32),
            pltpu.VMEM((_H, 16), jnp.int32),
            pltpu.VMEM((_H, _LANES, _D), jnp.float32),
            pltpu.VMEM((_H, _D, _LANES), jnp.float32),
        ] + [pltpu.SemaphoreType.DMA] * 9,
        compiler_params=pltpu.CompilerParams(use_tc_tiling_on_sc=False,
                                             needs_layout_passes=False),
    )
    out6 = run(ids3, offs16, embedding)
    return out6.transpose(0, 3, 5, 1, 2, 4).reshape(B, T, H, _D)
